# baseline (speedup 1.0000x reference)
import functools

import jax
import jax.numpy as jnp
from jax import lax
from jax.experimental import pallas as pl
from jax.experimental.pallas import tpu as pltpu
from jax.experimental.pallas import tpu_sc as plsc

NUM_NODES = 100000
FEAT_DIM = 128
EMB_DIM = 64
BATCH = 16384
OUT_DIM = FEAT_DIM + EMB_DIM

NC = 2
NS = 16
NW = NC * NS
BPW = BATCH // NW
NCHUNK = 4
C = BPW // NCHUNK
L = 16
G = C // L

NBLK = NUM_NODES // FEAT_DIM
NPAD = (NBLK + 1) * FEAT_DIM
TAIL = NUM_NODES - NBLK * FEAT_DIM
BLK_PER_W = (NBLK + NW - 1) // NW

_mesh = plsc.VectorSubcoreMesh(core_axis_name="c", subcore_axis_name="s")


@functools.partial(
    pl.kernel,
    mesh=_mesh,
    out_type=jax.ShapeDtypeStruct((NPAD, FEAT_DIM), jnp.float32),
    scratch_types=[
        pltpu.VMEM((2, EMB_DIM, FEAT_DIM), jnp.float32),
        pltpu.VMEM((2, FEAT_DIM, FEAT_DIM), jnp.float32),
        pltpu.SemaphoreType.DMA,
        pltpu.SemaphoreType.DMA,
    ],
    compiler_params=pltpu.CompilerParams(needs_layout_passes=False),
)
def _table_transpose(et_hbm, tail_hbm, embp_hbm, ebk, tbk, rsem, wsem):
    wid = lax.axis_index("s") * NC + lax.axis_index("c")
    lane = lax.iota(jnp.int32, L)

    @pl.when(wid == 0)
    def _():
        pltpu.sync_copy(tail_hbm, embp_hbm.at[pl.ds(NBLK * FEAT_DIM, TAIL)])

    def blk(i):
        nb = jnp.minimum(wid + NW * i, NBLK - 1)
        return pl.multiple_of(nb * FEAT_DIM, FEAT_DIM)

    def rd_copy(i, s):
        return pltpu.make_async_copy(
            et_hbm.at[pl.ds(0, EMB_DIM), pl.ds(blk(i), FEAT_DIM)],
            ebk.at[s], rsem)

    def wr_copy(i, s):
        return pltpu.make_async_copy(
            tbk.at[s], embp_hbm.at[pl.ds(blk(i), FEAT_DIM)], wsem)

    rd_copy(0, 0).start()

    def step(i, carry):
        s = i % 2
        rd_copy(i, s).wait()

        @pl.when(i + 1 < BLK_PER_W)
        def _():
            rd_copy(i + 1, 1 - s).start()

        for g in range(EMB_DIM // L):
            rvec = lane + (g * L)

            def tr(f, c2, rvec=rvec, s=s):
                cvec = (lane + f) & (FEAT_DIM - 1)
                v = plsc.load_gather(ebk.at[s], [rvec, cvec])
                plsc.store_scatter(tbk.at[s], [cvec, rvec], v)
                return c2

            lax.fori_loop(0, FEAT_DIM, tr, 0, unroll=8)

        @pl.when(i >= 2)
        def _():
            wr_copy(i - 2, s).wait()

        wr_copy(i, s).start()
        return carry

    lax.fori_loop(0, BLK_PER_W, step, 0)
    wr_copy(BLK_PER_W - 2, (BLK_PER_W - 2) % 2).wait()
    wr_copy(BLK_PER_W - 1, (BLK_PER_W - 1) % 2).wait()


def _dummy_read(i, pending):
    return pending


@functools.partial(
    pl.kernel,
    mesh=_mesh,
    out_type=jax.ShapeDtypeStruct((OUT_DIM, BATCH), jnp.float32),
    scratch_types=[
        pltpu.VMEM((BPW,), jnp.int32),
        pltpu.VMEM((2, C, FEAT_DIM), jnp.float32),
        pltpu.VMEM((2, C, FEAT_DIM), jnp.float32),
        pltpu.VMEM((2, FEAT_DIM, C), jnp.float32),
        pltpu.VMEM((2, EMB_DIM, C), jnp.float32),
        pltpu.SemaphoreType.DMA,
        pltpu.SemaphoreType.DMA,
        pltpu.SemaphoreType.DMA,
    ],
    compiler_params=pltpu.CompilerParams(needs_layout_passes=False),
)
def _encoder(idx_hbm, feat_hbm, embp_hbm, out_hbm, idx_v, fbuf, ebuf,
             tf, te, gsem0, gsem1, wsem):
    wid = lax.axis_index("s") * NC + lax.axis_index("c")
    base = wid * BPW
    pltpu.sync_copy(idx_hbm.at[pl.ds(base, BPW)], idx_v)

    gsems = (gsem0, gsem1)

    def start_gathers(j):
        s = j % 2
        ix = idx_v.at[pl.ds(j * C, C)]
        cf = pltpu.async_copy(feat_hbm.at[ix], fbuf.at[s], gsems[s])
        ce = pltpu.async_copy(embp_hbm.at[ix], ebuf.at[s], gsems[s])
        return cf, ce

    pending = start_gathers(0)
    writes = []
    lane = lax.iota(jnp.int32, L)
    for j in range(NCHUNK):
        s = j % 2
        cf, ce = pending
        cf.wait()
        ce.wait()
        if j + 1 < NCHUNK:
            pending = start_gathers(j + 1)

        for g in range(G):
            rvec = lane + (g * L)

            def tr_feat(f, carry, rvec=rvec, s=s):
                cvec = (lane + f) & (FEAT_DIM - 1)
                v = plsc.load_gather(fbuf.at[s], [rvec, cvec])
                plsc.store_scatter(tf.at[s], [cvec, rvec], v)
                return carry

            lax.fori_loop(0, FEAT_DIM, tr_feat, 0, unroll=8)

        for g in range(G):
            rvec = lane + (g * L)

            def tr_emb(f, carry, rvec=rvec, s=s):
                cvec = (lane + f) & (EMB_DIM - 1)
                v = plsc.load_gather(ebuf.at[s], [rvec, cvec])
                plsc.store_scatter(te.at[s], [cvec, rvec], v)
                return carry

            lax.fori_loop(0, EMB_DIM, tr_emb, 0, unroll=8)

        if j >= 2:
            for w in writes[j - 2]:
                w.wait()
        col = base + j * C
        wf = pltpu.async_copy(
            tf.at[s], out_hbm.at[pl.ds(0, FEAT_DIM), pl.ds(col, C)], wsem)
        we = pltpu.async_copy(
            te.at[s], out_hbm.at[pl.ds(FEAT_DIM, EMB_DIM), pl.ds(col, C)],
            wsem)
        writes.append((wf, we))

    for pair in writes[-2:]:
        for w in pair:
            w.wait()


def kernel(indices, features, emb_table):
    idx = indices.astype(jnp.int32)
    et = emb_table.T
    tail = jnp.pad(emb_table[NBLK * FEAT_DIM:, :],
                   ((0, 0), (0, FEAT_DIM - EMB_DIM)))
    embp = _table_transpose(et, tail)
    out_t = _encoder(idx, features, embp)
    return out_t.T
